# baseline (speedup 1.0000x reference)
import functools

import jax
import jax.numpy as jnp
from jax import lax
from jax.experimental import pallas as pl
from jax.experimental.pallas import tpu as pltpu
from jax.experimental.pallas import tpu_sc as plsc

NP_, ND, E, D = 50000, 10000, 320000, 128
NC, NS = 2, 16
NW = NC * NS
EW = E // NW
EB = 80
NBLK = EW // EB

CHUNKS_USE = ((0, 10000),)
CHUNKS_REV = ((0, 12512), (12512, 12512), (25024, 12512), (37536, 12464))


def _seg_kernel_body(chunks, num_dst, with_cnt, full_range, *refs):
    if with_cnt:
        (x_hbm, src_hbm, dst_hbm, zero_hbm, zero8_hbm, ones_hbm,
         out_hbm, cnt_hbm,
         sidx_v, didx_v, rows_v, ones_v,
         acc_sh, cnt_sh, sem) = refs
    else:
        (x_hbm, src_hbm, dst_hbm, zero_hbm,
         out_hbm,
         sidx_v, didx_v, rows_v,
         acc_sh, sem) = refs
        zero8_hbm = ones_hbm = cnt_hbm = ones_v = cnt_sh = None

    c = lax.axis_index("c")
    s = lax.axis_index("s")
    w = c * NS + s
    edge_base = w * EW

    if with_cnt:
        pltpu.sync_copy(ones_hbm, ones_v)

    for (lo, rsz) in chunks:
        rz = (rsz + 16) // 16
        rw = rsz // 16
        pltpu.sync_copy(zero_hbm.at[pl.ds(0, rz)], acc_sh.at[pl.ds(s * rz, rz)])
        if with_cnt:
            pltpu.sync_copy(zero8_hbm.at[pl.ds(0, rz)],
                            cnt_sh.at[pl.ds(s * rz, rz)])
        plsc.subcore_barrier()

        def block(i, carry, lo=lo, rsz=rsz):
            base = edge_base + i * EB
            pltpu.sync_copy(src_hbm.at[pl.ds(base, EB)], sidx_v)
            pltpu.sync_copy(dst_hbm.at[pl.ds(base, EB)], didx_v)
            if not full_range:
                for t in range(EB // 16):
                    sv = sidx_v[pl.ds(16 * t, 16)]
                    dv = didx_v[pl.ds(16 * t, 16)]
                    inr = (dv >= lo) & (dv < lo + rsz)
                    didx_v[pl.ds(16 * t, 16)] = jnp.where(inr, dv - lo, rsz)
                    sidx_v[pl.ds(16 * t, 16)] = jnp.where(inr, sv, 0)
            pltpu.async_copy(x_hbm.at[sidx_v], rows_v, sem).wait()
            pltpu.sync_copy(rows_v, acc_sh.at[didx_v], add=True)
            if with_cnt:
                pltpu.sync_copy(ones_v, cnt_sh.at[didx_v], add=True)
            return carry

        lax.fori_loop(0, NBLK, block, 0)
        plsc.subcore_barrier()

        pltpu.sync_copy(acc_sh.at[pl.ds(s * rw, rw)],
                        out_hbm.at[c, pl.ds(lo + s * rw, rw)])
        if with_cnt:
            pltpu.sync_copy(cnt_sh.at[pl.ds(s * rw, rw)],
                            cnt_hbm.at[c, pl.ds(lo + s * rw, rw)])
        if len(chunks) > 1:
            plsc.subcore_barrier()


@functools.cache
def _make_seg_kernel(chunks, num_dst, with_cnt, full_range):
    max_r = max(r for _, r in chunks)
    mesh = plsc.VectorSubcoreMesh(core_axis_name="c", subcore_axis_name="s")
    out_type = [jax.ShapeDtypeStruct((NC, num_dst, D), jnp.float32)]
    scratch = [
        pltpu.VMEM((EB,), jnp.int32),
        pltpu.VMEM((EB,), jnp.int32),
        pltpu.VMEM((EB, D), jnp.float32),
    ]
    if with_cnt:
        out_type.append(jax.ShapeDtypeStruct((NC, num_dst, 8), jnp.float32))
        scratch.append(pltpu.VMEM((EB, 8), jnp.float32))
    scratch.append(pltpu.VMEM_SHARED((max_r + 16, D), jnp.float32))
    if with_cnt:
        scratch.append(pltpu.VMEM_SHARED((max_r + 16, 8), jnp.float32))
    scratch.append(pltpu.SemaphoreType.DMA)
    body = functools.partial(_seg_kernel_body, chunks, num_dst, with_cnt,
                             full_range)
    return pl.kernel(body, out_type=out_type, mesh=mesh,
                     scratch_types=scratch,
                     compiler_params=pltpu.CompilerParams(
                         use_tc_tiling_on_sc=False))


def _seg_sum(x_src, src, dst, chunks, num_dst, with_cnt):
    full_range = len(chunks) == 1 and chunks[0][1] >= num_dst
    max_r = max(r for _, r in chunks)
    rz_max = (max_r + 16) // 16
    k = _make_seg_kernel(tuple(chunks), num_dst, with_cnt, full_range)
    zero = jnp.zeros((rz_max, D), jnp.float32)
    if with_cnt:
        zero8 = jnp.zeros((rz_max, 8), jnp.float32)
        ones = jnp.ones((EB, 8), jnp.float32)
        return k(x_src, src, dst, zero, zero8, ones)
    return k(x_src, src, dst, zero)[0]


def _fuse_body(s0, s1, c0, c1, xd, W, root, b, g, beta, y):
    cnt = c0[:, 0:1] + c1[:, 0:1]
    inv = 1.0 / jnp.maximum(cnt, 1.0)
    h = (s0[:, :] + s1[:, :]) * inv
    out = (jnp.dot(h, W[:, :], preferred_element_type=jnp.float32)
           + jnp.dot(xd[:, :], root[:, :], preferred_element_type=jnp.float32)
           + b[:, :])
    r = jnp.maximum(out + xd[:, :], 0.0)
    mu = jnp.mean(r, axis=-1, keepdims=True)
    var = jnp.mean((r - mu) ** 2, axis=-1, keepdims=True)
    y[:, :] = (r - mu) * lax.rsqrt(var + 1e-5) * g[:, :] + beta[:, :]


@functools.cache
def _make_fuse(n, blk=512):
    grid = (n + blk - 1) // blk
    row = pl.BlockSpec((blk, D), lambda i: (i, 0))
    row8 = pl.BlockSpec((blk, 8), lambda i: (i, 0))
    full = pl.BlockSpec((D, D), lambda i: (0, 0))
    vec = pl.BlockSpec((1, D), lambda i: (0, 0))
    return pl.pallas_call(
        _fuse_body,
        grid=(grid,),
        in_specs=[row, row, row8, row8, row, full, full, vec, vec, vec],
        out_specs=row,
        out_shape=jax.ShapeDtypeStruct((n, D), jnp.float32),
    )


def _fuse(s, cnt, xd, W, root, b, g, beta):
    n = xd.shape[0]
    return _make_fuse(n)(s[0], s[1], cnt[0], cnt[1], xd, W, root,
                         b.reshape(1, D), g.reshape(1, D),
                         beta.reshape(1, D))


def kernel(x_paper, x_dataset, edge_index_use, edge_index_rev,
           W1_use, root1_use, b1_use, W1_rev, root1_rev, b1_rev,
           W2_use, root2_use, b2_use, W2_rev, root2_rev, b2_rev,
           g1, beta1, g2, beta2):
    su = edge_index_use[0].astype(jnp.int32)
    du = edge_index_use[1].astype(jnp.int32)
    sr = edge_index_rev[0].astype(jnp.int32)
    dr = edge_index_rev[1].astype(jnp.int32)

    s_d, c_d = _seg_sum(x_paper, su, du, CHUNKS_USE, ND, True)
    x_dataset_b, _ = lax.optimization_barrier((x_dataset, s_d))
    s_p, c_p = _seg_sum(x_dataset_b, sr, dr, CHUNKS_REV, NP_, True)
    xd1 = _fuse(s_d, c_d, x_dataset, W1_use[0], root1_use, b1_use, g1, beta1)
    xp1 = _fuse(s_p, c_p, x_paper, W1_rev[1], root1_rev, b1_rev, g1, beta1)

    s_d2 = _seg_sum(xp1, su, du, CHUNKS_USE, ND, False)
    xd1_b, _ = lax.optimization_barrier((xd1, s_d2))
    s_p2 = _seg_sum(xd1_b, sr, dr, CHUNKS_REV, NP_, False)
    xd2 = _fuse(s_d2, c_d, xd1, W2_use[0], root2_use, b2_use, g2, beta2)
    xp2 = _fuse(s_p2, c_p, xp1, W2_rev[1], root2_rev, b2_rev, g2, beta2)
    return (xp2, xd2)


# speedup vs baseline: 1.0000x; 1.0000x over previous
import functools

import jax
import jax.numpy as jnp
from jax import lax
from jax.experimental import pallas as pl
from jax.experimental.pallas import tpu as pltpu
from jax.experimental.pallas import tpu_sc as plsc

NP_, ND, E, D = 50000, 10000, 320000, 128
NC, NS = 2, 16
NW = NC * NS
EW = E // NW
EB = 80
NBLK = EW // EB

CHUNKS_USE = ((0, 10000),)
CHUNKS_REV = ((0, 12512), (12512, 12512), (25024, 12512), (37536, 12464))


def _seg_kernel_body(chunks, num_dst, with_cnt, full_range, *refs):
    if with_cnt:
        (x_hbm, src_hbm, dst_hbm, zero_hbm, zero8_hbm, ones_hbm,
         out_hbm, cnt_hbm,
         sidx_v, didx_v, rows_v, ones_v,
         acc_sh, cnt_sh, sem) = refs
    else:
        (x_hbm, src_hbm, dst_hbm, zero_hbm,
         out_hbm,
         sidx_v, didx_v, rows_v,
         acc_sh, sem) = refs
        zero8_hbm = ones_hbm = cnt_hbm = ones_v = cnt_sh = None

    c = lax.axis_index("c")
    s = lax.axis_index("s")
    w = c * NS + s
    edge_base = w * EW

    if with_cnt:
        pltpu.sync_copy(ones_hbm, ones_v)

    for (lo, rsz) in chunks:
        rz = (rsz + 16) // 16
        rw = rsz // 16
        pltpu.sync_copy(zero_hbm.at[pl.ds(0, rz)], acc_sh.at[pl.ds(s * rz, rz)])
        if with_cnt:
            pltpu.sync_copy(zero8_hbm.at[pl.ds(0, rz)],
                            cnt_sh.at[pl.ds(s * rz, rz)])
        plsc.subcore_barrier()

        dummy = rsz + s * 16 + lax.iota(jnp.int32, 16)

        def block(i, carry, lo=lo, rsz=rsz, dummy=dummy):
            base = edge_base + i * EB
            pltpu.sync_copy(src_hbm.at[pl.ds(base, EB)], sidx_v)
            pltpu.sync_copy(dst_hbm.at[pl.ds(base, EB)], didx_v)
            if not full_range:
                for t in range(EB // 16):
                    sv = sidx_v[pl.ds(16 * t, 16)]
                    dv = didx_v[pl.ds(16 * t, 16)]
                    inr = (dv >= lo) & (dv < lo + rsz)
                    didx_v[pl.ds(16 * t, 16)] = jnp.where(inr, dv - lo, dummy)
                    sidx_v[pl.ds(16 * t, 16)] = jnp.where(inr, sv, 0)
            pltpu.async_copy(x_hbm.at[sidx_v], rows_v, sem).wait()
            pltpu.sync_copy(rows_v, acc_sh.at[didx_v], add=True)
            if with_cnt:
                pltpu.sync_copy(ones_v, cnt_sh.at[didx_v], add=True)
            return carry

        lax.fori_loop(0, NBLK, block, 0)
        plsc.subcore_barrier()

        pltpu.sync_copy(acc_sh.at[pl.ds(s * rw, rw)],
                        out_hbm.at[c, pl.ds(lo + s * rw, rw)])
        if with_cnt:
            pltpu.sync_copy(cnt_sh.at[pl.ds(s * rw, rw)],
                            cnt_hbm.at[c, pl.ds(lo + s * rw, rw)])
        if len(chunks) > 1:
            plsc.subcore_barrier()


@functools.cache
def _make_seg_kernel(chunks, num_dst, with_cnt, full_range):
    max_r = max(r for _, r in chunks)
    mesh = plsc.VectorSubcoreMesh(core_axis_name="c", subcore_axis_name="s")
    out_type = [jax.ShapeDtypeStruct((NC, num_dst, D), jnp.float32)]
    scratch = [
        pltpu.VMEM((EB,), jnp.int32),
        pltpu.VMEM((EB,), jnp.int32),
        pltpu.VMEM((EB, D), jnp.float32),
    ]
    if with_cnt:
        out_type.append(jax.ShapeDtypeStruct((NC, num_dst, 8), jnp.float32))
        scratch.append(pltpu.VMEM((EB, 8), jnp.float32))
    scratch.append(pltpu.VMEM_SHARED((max_r + 16 * (NS + 1), D),
                                     jnp.float32))
    if with_cnt:
        scratch.append(pltpu.VMEM_SHARED((max_r + 16 * (NS + 1), 8),
                                         jnp.float32))
    scratch.append(pltpu.SemaphoreType.DMA)
    body = functools.partial(_seg_kernel_body, chunks, num_dst, with_cnt,
                             full_range)
    return pl.kernel(body, out_type=out_type, mesh=mesh,
                     scratch_types=scratch,
                     compiler_params=pltpu.CompilerParams(
                         use_tc_tiling_on_sc=False))


def _seg_sum(x_src, src, dst, chunks, num_dst, with_cnt):
    full_range = len(chunks) == 1 and chunks[0][1] >= num_dst
    max_r = max(r for _, r in chunks)
    rz_max = (max_r + 16) // 16
    k = _make_seg_kernel(tuple(chunks), num_dst, with_cnt, full_range)
    zero = jnp.zeros((rz_max, D), jnp.float32)
    if with_cnt:
        zero8 = jnp.zeros((rz_max, 8), jnp.float32)
        ones = jnp.ones((EB, 8), jnp.float32)
        return k(x_src, src, dst, zero, zero8, ones)
    return k(x_src, src, dst, zero)[0]


def _fuse_body(s0, s1, c0, c1, xd, W, root, b, g, beta, y):
    cnt = c0[:, 0:1] + c1[:, 0:1]
    inv = 1.0 / jnp.maximum(cnt, 1.0)
    h = (s0[:, :] + s1[:, :]) * inv
    out = (jnp.dot(h, W[:, :], preferred_element_type=jnp.float32)
           + jnp.dot(xd[:, :], root[:, :], preferred_element_type=jnp.float32)
           + b[:, :])
    r = jnp.maximum(out + xd[:, :], 0.0)
    mu = jnp.mean(r, axis=-1, keepdims=True)
    var = jnp.mean((r - mu) ** 2, axis=-1, keepdims=True)
    y[:, :] = (r - mu) * lax.rsqrt(var + 1e-5) * g[:, :] + beta[:, :]


@functools.cache
def _make_fuse(n, blk=512):
    grid = (n + blk - 1) // blk
    row = pl.BlockSpec((blk, D), lambda i: (i, 0))
    row8 = pl.BlockSpec((blk, 8), lambda i: (i, 0))
    full = pl.BlockSpec((D, D), lambda i: (0, 0))
    vec = pl.BlockSpec((1, D), lambda i: (0, 0))
    return pl.pallas_call(
        _fuse_body,
        grid=(grid,),
        in_specs=[row, row, row8, row8, row, full, full, vec, vec, vec],
        out_specs=row,
        out_shape=jax.ShapeDtypeStruct((n, D), jnp.float32),
    )


def _fuse(s, cnt, xd, W, root, b, g, beta):
    n = xd.shape[0]
    return _make_fuse(n)(s[0], s[1], cnt[0], cnt[1], xd, W, root,
                         b.reshape(1, D), g.reshape(1, D),
                         beta.reshape(1, D))


def kernel(x_paper, x_dataset, edge_index_use, edge_index_rev,
           W1_use, root1_use, b1_use, W1_rev, root1_rev, b1_rev,
           W2_use, root2_use, b2_use, W2_rev, root2_rev, b2_rev,
           g1, beta1, g2, beta2):
    su = edge_index_use[0].astype(jnp.int32)
    du = edge_index_use[1].astype(jnp.int32)
    sr = edge_index_rev[0].astype(jnp.int32)
    dr = edge_index_rev[1].astype(jnp.int32)

    s_d, c_d = _seg_sum(x_paper, su, du, CHUNKS_USE, ND, True)
    x_dataset_b, _ = lax.optimization_barrier((x_dataset, s_d))
    s_p, c_p = _seg_sum(x_dataset_b, sr, dr, CHUNKS_REV, NP_, True)
    xd1 = _fuse(s_d, c_d, x_dataset, W1_use[0], root1_use, b1_use, g1, beta1)
    xp1 = _fuse(s_p, c_p, x_paper, W1_rev[1], root1_rev, b1_rev, g1, beta1)

    s_d2 = _seg_sum(xp1, su, du, CHUNKS_USE, ND, False)
    xd1_b, _ = lax.optimization_barrier((xd1, s_d2))
    s_p2 = _seg_sum(xd1_b, sr, dr, CHUNKS_REV, NP_, False)
    xd2 = _fuse(s_d2, c_d, xd1, W2_use[0], root2_use, b2_use, g2, beta2)
    xp2 = _fuse(s_p2, c_p, xp1, W2_rev[1], root2_rev, b2_rev, g2, beta2)
    return (xp2, xd2)


# speedup vs baseline: 20.7833x; 20.7829x over previous
import functools

import jax
import jax.numpy as jnp
from jax import lax
from jax.experimental import pallas as pl
from jax.experimental.pallas import tpu as pltpu
from jax.experimental.pallas import tpu_sc as plsc

NP_, ND, E, D = 50000, 10000, 320000, 128
NC, NS = 2, 16
NW = NC * NS
EW = E // NW
EB = 80
NBLK = EW // EB

CHUNKS_USE = ((0, 10000),)
CHUNKS_REV = ((0, 12512), (12512, 12512), (25024, 12512), (37536, 12464))


def _seg_kernel_body(chunks, num_dst, with_cnt, full_range, *refs):
    if with_cnt:
        (x_hbm, src_hbm, dst_hbm, zero_hbm, zero8_hbm, ones_hbm,
         out_hbm, cnt_hbm,
         sidx_v, didx_v, rows_v, ones_v,
         acc_sh, cnt_sh, sem) = refs
    else:
        (x_hbm, src_hbm, dst_hbm, zero_hbm,
         out_hbm,
         sidx_v, didx_v, rows_v,
         acc_sh, sem) = refs
        zero8_hbm = ones_hbm = cnt_hbm = ones_v = cnt_sh = None

    c = lax.axis_index("c")
    s = lax.axis_index("s")
    w = c * NS + s
    edge_base = w * EW

    if with_cnt:
        pltpu.sync_copy(ones_hbm, ones_v)

    for (lo, rsz) in chunks:
        rz = (rsz + 16) // 16
        rw = rsz // 16
        pltpu.sync_copy(zero_hbm.at[pl.ds(0, rz)], acc_sh.at[pl.ds(s * rz, rz)])
        if with_cnt:
            pltpu.sync_copy(zero8_hbm.at[pl.ds(0, rz)],
                            cnt_sh.at[pl.ds(s * rz, rz)])
        plsc.subcore_barrier()

        dummy = rsz + s * 16 + lax.iota(jnp.int32, 16)

        def block(i, carry, lo=lo, rsz=rsz, dummy=dummy):
            base = edge_base + i * EB
            pltpu.sync_copy(src_hbm.at[pl.ds(base, EB)], sidx_v)
            pltpu.sync_copy(dst_hbm.at[pl.ds(base, EB)], didx_v)
            if not full_range:
                for t in range(EB // 16):
                    dv = didx_v[pl.ds(16 * t, 16)]
                    inr = (dv >= lo) & (dv < lo + rsz)
                    didx_v[pl.ds(16 * t, 16)] = jnp.where(inr, dv - lo, dummy)
            pltpu.async_copy(x_hbm.at[sidx_v], rows_v, sem).wait()
            pltpu.sync_copy(rows_v, acc_sh.at[didx_v], add=True)
            if with_cnt:
                pltpu.sync_copy(ones_v, cnt_sh.at[didx_v], add=True)
            return carry

        lax.fori_loop(0, NBLK, block, 0)
        plsc.subcore_barrier()

        pltpu.sync_copy(acc_sh.at[pl.ds(s * rw, rw)],
                        out_hbm.at[c, pl.ds(lo + s * rw, rw)])
        if with_cnt:
            pltpu.sync_copy(cnt_sh.at[pl.ds(s * rw, rw)],
                            cnt_hbm.at[c, pl.ds(lo + s * rw, rw)])
        if len(chunks) > 1:
            plsc.subcore_barrier()


@functools.cache
def _make_seg_kernel(chunks, num_dst, with_cnt, full_range):
    max_r = max(r for _, r in chunks)
    mesh = plsc.VectorSubcoreMesh(core_axis_name="c", subcore_axis_name="s")
    out_type = [jax.ShapeDtypeStruct((NC, num_dst, D), jnp.float32)]
    scratch = [
        pltpu.VMEM((EB,), jnp.int32),
        pltpu.VMEM((EB,), jnp.int32),
        pltpu.VMEM((EB, D), jnp.float32),
    ]
    if with_cnt:
        out_type.append(jax.ShapeDtypeStruct((NC, num_dst, 8), jnp.float32))
        scratch.append(pltpu.VMEM((EB, 8), jnp.float32))
    scratch.append(pltpu.VMEM_SHARED((max_r + 16 * (NS + 1), D),
                                     jnp.float32))
    if with_cnt:
        scratch.append(pltpu.VMEM_SHARED((max_r + 16 * (NS + 1), 8),
                                         jnp.float32))
    scratch.append(pltpu.SemaphoreType.DMA)
    body = functools.partial(_seg_kernel_body, chunks, num_dst, with_cnt,
                             full_range)
    return pl.kernel(body, out_type=out_type, mesh=mesh,
                     scratch_types=scratch,
                     compiler_params=pltpu.CompilerParams(
                         use_tc_tiling_on_sc=False))


def _seg_sum(x_src, src, dst, chunks, num_dst, with_cnt):
    full_range = len(chunks) == 1 and chunks[0][1] >= num_dst
    max_r = max(r for _, r in chunks)
    rz_max = (max_r + 16) // 16
    k = _make_seg_kernel(tuple(chunks), num_dst, with_cnt, full_range)
    zero = jnp.zeros((rz_max, D), jnp.float32)
    if with_cnt:
        zero8 = jnp.zeros((rz_max, 8), jnp.float32)
        ones = jnp.ones((EB, 8), jnp.float32)
        return k(x_src, src, dst, zero, zero8, ones)
    return k(x_src, src, dst, zero)[0]


def _fuse_body(s0, s1, c0, c1, xd, W, root, b, g, beta, y):
    cnt = c0[:, 0:1] + c1[:, 0:1]
    inv = 1.0 / jnp.maximum(cnt, 1.0)
    h = (s0[:, :] + s1[:, :]) * inv
    out = (jnp.dot(h, W[:, :], preferred_element_type=jnp.float32)
           + jnp.dot(xd[:, :], root[:, :], preferred_element_type=jnp.float32)
           + b[:, :])
    r = jnp.maximum(out + xd[:, :], 0.0)
    mu = jnp.mean(r, axis=-1, keepdims=True)
    var = jnp.mean((r - mu) ** 2, axis=-1, keepdims=True)
    y[:, :] = (r - mu) * lax.rsqrt(var + 1e-5) * g[:, :] + beta[:, :]


@functools.cache
def _make_fuse(n, blk=512):
    grid = (n + blk - 1) // blk
    row = pl.BlockSpec((blk, D), lambda i: (i, 0))
    row8 = pl.BlockSpec((blk, 8), lambda i: (i, 0))
    full = pl.BlockSpec((D, D), lambda i: (0, 0))
    vec = pl.BlockSpec((1, D), lambda i: (0, 0))
    return pl.pallas_call(
        _fuse_body,
        grid=(grid,),
        in_specs=[row, row, row8, row8, row, full, full, vec, vec, vec],
        out_specs=row,
        out_shape=jax.ShapeDtypeStruct((n, D), jnp.float32),
    )


def _fuse(s, cnt, xd, W, root, b, g, beta):
    n = xd.shape[0]
    return _make_fuse(n)(s[0], s[1], cnt[0], cnt[1], xd, W, root,
                         b.reshape(1, D), g.reshape(1, D),
                         beta.reshape(1, D))


def kernel(x_paper, x_dataset, edge_index_use, edge_index_rev,
           W1_use, root1_use, b1_use, W1_rev, root1_rev, b1_rev,
           W2_use, root2_use, b2_use, W2_rev, root2_rev, b2_rev,
           g1, beta1, g2, beta2):
    su = edge_index_use[0].astype(jnp.int32)
    du = edge_index_use[1].astype(jnp.int32)
    sr = edge_index_rev[0].astype(jnp.int32)
    dr = edge_index_rev[1].astype(jnp.int32)

    s_d, c_d = _seg_sum(x_paper, su, du, CHUNKS_USE, ND, True)
    x_dataset_b, _ = lax.optimization_barrier((x_dataset, s_d))
    s_p, c_p = _seg_sum(x_dataset_b, sr, dr, CHUNKS_REV, NP_, True)
    xd1 = _fuse(s_d, c_d, x_dataset, W1_use[0], root1_use, b1_use, g1, beta1)
    xp1 = _fuse(s_p, c_p, x_paper, W1_rev[1], root1_rev, b1_rev, g1, beta1)

    s_d2 = _seg_sum(xp1, su, du, CHUNKS_USE, ND, False)
    xd1_b, _ = lax.optimization_barrier((xd1, s_d2))
    s_p2 = _seg_sum(xd1_b, sr, dr, CHUNKS_REV, NP_, False)
    xd2 = _fuse(s_d2, c_d, xd1, W2_use[0], root2_use, b2_use, g2, beta2)
    xp2 = _fuse(s_p2, c_p, xp1, W2_rev[1], root2_rev, b2_rev, g2, beta2)
    return (xp2, xd2)
